# baseline (speedup 1.0000x reference)
import functools

import jax
import jax.numpy as jnp
from jax import lax
from jax.experimental import pallas as pl
from jax.experimental.pallas import tpu as pltpu
from jax.experimental.pallas import tpu_sc as plsc


def _tc_fold_actions(act_emb_self, act_emb_other, W1, b1):
    NA, D = act_emb_self.shape

    def body(aes_ref, aeo_ref, w1_ref, b1_ref, pb_ref, pc_ref):
        pb_ref[...] = jnp.dot(aes_ref[...], w1_ref[D:2 * D, :],
                              preferred_element_type=jnp.float32) + b1_ref[...]
        pc_ref[...] = jnp.dot(aeo_ref[...], w1_ref[2 * D:3 * D, :],
                              preferred_element_type=jnp.float32)

    t = jax.ShapeDtypeStruct((NA, D), jnp.float32)
    return pl.pallas_call(
        body,
        out_shape=[t, t],
    )(act_emb_self, act_emb_other, W1, b1.reshape(1, -1))


def _sc_gather_state(state_emb, sidx):
    B = sidx.shape[0]
    D = state_emb.shape[1]
    info = plsc.get_sparse_core_info()
    NC, NS = info.num_cores, info.num_subcores
    NW = NC * NS
    assert B % (16 * NW) == 0
    b_per_w = B // NW
    ch = b_per_w // 2

    mesh = plsc.VectorSubcoreMesh(core_axis_name="c", subcore_axis_name="s")

    @functools.partial(
        pl.kernel,
        mesh=mesh,
        out_type=jax.ShapeDtypeStruct((B, D), jnp.float32),
        scratch_types=[
            pltpu.VMEM((b_per_w,), jnp.int32),
            pltpu.VMEM((ch, D), jnp.float32),
            pltpu.VMEM((ch, D), jnp.float32),
            pltpu.SemaphoreType.DMA,
            pltpu.SemaphoreType.DMA,
            pltpu.SemaphoreType.DMA,
            pltpu.SemaphoreType.DMA,
        ],
    )
    def gather_k(table_hbm, sidx_hbm, out_s, idx_v, buf0, buf1,
                 g0, g1, w0, w1):
        wid = lax.axis_index("s") * NC + lax.axis_index("c")
        base = wid * b_per_w
        bufs, gsems, wsems = [buf0, buf1], [g0, g1], [w0, w1]

        pltpu.sync_copy(sidx_hbm.at[pl.ds(base, b_per_w)], idx_v)

        gathers = [
            pltpu.make_async_copy(
                table_hbm.at[idx_v.at[pl.ds(c * ch, ch)]], bufs[c], gsems[c])
            for c in range(2)
        ]
        for g in gathers:
            g.start()
        writes = [None, None]
        for c in range(2):
            gathers[c].wait()
            writes[c] = pltpu.make_async_copy(
                bufs[c], out_s.at[pl.ds(base + c * ch, ch)], wsems[c])
            writes[c].start()
        for c in range(2):
            writes[c].wait()

    return gather_k(state_emb, sidx)


def _sc_gather_actions_sum(PB, PC, aidx, oidx):
    B = aidx.shape[0]
    D = PB.shape[1]
    info = plsc.get_sparse_core_info()
    NC, NS = info.num_cores, info.num_subcores
    NW = NC * NS
    b_per_w = B // NW
    nch = 4
    ch = b_per_w // nch

    mesh = plsc.VectorSubcoreMesh(core_axis_name="c", subcore_axis_name="s")

    @functools.partial(
        pl.kernel,
        mesh=mesh,
        out_type=jax.ShapeDtypeStruct((B, D), jnp.float32),
        scratch_types=[
            pltpu.VMEM((2 * b_per_w,), jnp.int32),
            pltpu.VMEM((ch, D), jnp.float32),
            pltpu.VMEM((ch, D), jnp.float32),
            pltpu.VMEM((ch, D), jnp.float32),
            pltpu.VMEM((ch, D), jnp.float32),
            pltpu.SemaphoreType.DMA,
            pltpu.SemaphoreType.DMA,
            pltpu.SemaphoreType.DMA,
            pltpu.SemaphoreType.DMA,
            pltpu.SemaphoreType.DMA,
            pltpu.SemaphoreType.DMA,
        ],
    )
    def gather_k(pb_hbm, pc_hbm, aidx_hbm, oidx_hbm, out_ap, idx_v,
                 pb0, pc0, pb1, pc1, ga0, gc0, ga1, gc1, w0, w1):
        wid = lax.axis_index("s") * NC + lax.axis_index("c")
        base = wid * b_per_w
        pbufs, cbufs = [pb0, pb1], [pc0, pc1]
        gasems, gcsems, wsems = [ga0, ga1], [gc0, gc1], [w0, w1]

        pltpu.sync_copy(aidx_hbm.at[pl.ds(base, b_per_w)],
                        idx_v.at[pl.ds(0, b_per_w)])
        pltpu.sync_copy(oidx_hbm.at[pl.ds(base, b_per_w)],
                        idx_v.at[pl.ds(b_per_w, b_per_w)])

        def start_gathers(c):
            p = c % 2
            ga = pltpu.make_async_copy(
                pb_hbm.at[idx_v.at[pl.ds(c * ch, ch)]], pbufs[p], gasems[p])
            gc = pltpu.make_async_copy(
                pc_hbm.at[idx_v.at[pl.ds(b_per_w + c * ch, ch)]],
                cbufs[p], gcsems[p])
            ga.start()
            gc.start()
            return ga, gc

        def add_bufs(dst, src):
            def row(i, carry):
                for j in range(D // 16):
                    sl = pl.ds(j * 16, 16)
                    dst[i, sl] = dst[i, sl] + src[i, sl]
                return carry
            lax.fori_loop(0, ch, row, 0)

        gathers = [None] * nch
        writes = [None] * nch
        gathers[0] = start_gathers(0)
        gathers[1] = start_gathers(1)
        for c in range(nch):
            p = c % 2
            gathers[c][0].wait()
            gathers[c][1].wait()
            add_bufs(pbufs[p], cbufs[p])
            writes[c] = pltpu.make_async_copy(
                pbufs[p], out_ap.at[pl.ds(base + c * ch, ch)], wsems[p])
            writes[c].start()
            if c + 2 < nch:
                writes[c].wait()
                gathers[c + 2] = start_gathers(c + 2)
        for c in range(nch - 2, nch):
            writes[c].wait()

    return gather_k(PB, PC, aidx, oidx)


def _tc_mlp(S, AP, W1, W2, b2, block_m):
    B, D = S.shape
    n_blocks = B // block_m

    def body(s_ref, ap_ref, w1_ref, w2_ref, b2_ref, out_ref):
        h = jnp.dot(s_ref[...], w1_ref[0:D, :],
                    preferred_element_type=jnp.float32)
        h = jnp.maximum(h + ap_ref[...], 0.0)
        r = lax.dot_general(w2_ref[...], h, (((0,), (1,)), ((), ())),
                            preferred_element_type=jnp.float32)
        out_ref[...] = r + b2_ref[0]

    out = pl.pallas_call(
        body,
        grid=(n_blocks,),
        in_specs=[
            pl.BlockSpec((block_m, D), lambda i: (i, 0)),
            pl.BlockSpec((block_m, D), lambda i: (i, 0)),
            pl.BlockSpec((3 * D, D), lambda i: (0, 0)),
            pl.BlockSpec((D, 1), lambda i: (0, 0)),
            pl.BlockSpec(memory_space=pltpu.SMEM),
        ],
        out_specs=pl.BlockSpec((1, block_m), lambda i: (0, i)),
        out_shape=jax.ShapeDtypeStruct((1, B), jnp.float32),
        compiler_params=pltpu.CompilerParams(
            dimension_semantics=("arbitrary",),
        ),
    )(S, AP, W1, W2, b2)
    return out.reshape(B)


def kernel(state_indices, joint_actions, state_emb, act_emb_self,
           act_emb_other, W1, b1, W2, b2):
    sidx = state_indices.astype(jnp.int32)
    aidx = joint_actions[:, 0].astype(jnp.int32)
    oidx = joint_actions[:, 1].astype(jnp.int32)

    PB, PC = _tc_fold_actions(act_emb_self, act_emb_other, W1, b1)
    S = _sc_gather_state(state_emb, sidx)
    AP = _sc_gather_actions_sum(PB, PC, aidx, oidx)

    return _tc_mlp(S, AP, W1, W2, b2, block_m=2048)
